# baseline (speedup 1.0000x reference)
import functools

import numpy as np

import jax
import jax.numpy as jnp
from jax import lax
from jax.experimental import pallas as pl
from jax.experimental.pallas import tpu as pltpu
from jax.experimental.pallas import tpu_sc as plsc

_NOISE_ALPHA = 5.0


@functools.lru_cache(maxsize=None)
def _sc_gather_fn(V, D, B):
    info = plsc.get_sparse_core_info()
    NC, NS = info.num_cores, info.num_subcores
    NW = NC * NS
    JROWS = B // 128
    assert JROWS % NW == 0
    j_per_w = JROWS // NW
    CH = 8
    assert j_per_w % CH == 0
    n_iter = j_per_w // CH

    mesh = plsc.VectorSubcoreMesh(core_axis_name="c", subcore_axis_name="s")

    @functools.partial(
        pl.kernel,
        out_type=jax.ShapeDtypeStruct((B, D), jnp.float32),
        mesh=mesh,
        compiler_params=pltpu.CompilerParams(use_tc_tiling_on_sc=False),
        scratch_types=[
            pltpu.VMEM((j_per_w, 128), jnp.int32),
            pltpu.VMEM((CH * 128, D), jnp.float32),
            pltpu.SemaphoreType.DMA,
        ],
    )
    def sc_gather(table_hbm, idx_hbm, out_hbm, idx_v, rows_v, sem):
        wid = lax.axis_index("s") * NC + lax.axis_index("c")
        j0 = wid * j_per_w
        pltpu.sync_copy(idx_hbm.at[pl.ds(j0, j_per_w)], idx_v)
        out_base = j0 * 128

        def body(t, carry):
            js = t * CH
            copies = []
            for b in range(CH):
                copies.append(pltpu.async_copy(
                    table_hbm.at[idx_v.at[js + b]],
                    rows_v.at[pl.ds(b * 128, 128)],
                    sem,
                ))
            for c in copies:
                c.wait()
            pltpu.sync_copy(
                rows_v, out_hbm.at[pl.ds(out_base + t * (CH * 128), CH * 128)])
            return carry

        lax.fori_loop(0, n_iter, body, 0)

    return sc_gather


_ROTS = ((13, 15, 26, 6), (17, 29, 16, 24))


def _rotl(x, d):
    return (x << jnp.uint32(d)) | (x >> jnp.uint32(32 - d))


def _threefry2x32(x0, x1, ks):
    x0 = x0 + ks[0]
    x1 = x1 + ks[1]
    for i in range(5):
        for r in _ROTS[i % 2]:
            x0 = x0 + x1
            x1 = _rotl(x1, r) ^ x0
        x0 = x0 + ks[(i + 1) % 3]
        x1 = x1 + ks[(i + 2) % 3] + jnp.uint32(i + 1)
    return x0, x1


def _uniform_from_bits(bits, minval, scale):
    fb = (bits >> jnp.uint32(9)) | jnp.uint32(0x3F800000)
    f = lax.bitcast_convert_type(fb, jnp.float32) - jnp.float32(1.0)
    return f * jnp.float32(scale) + jnp.float32(minval)


def _noise_body(o_ref, *, br, ncols, rmul, cmul, minval, scale, poff=0):
    k1 = jnp.uint32(0)
    k2 = jnp.uint32(42)
    ks = (k1, k2, k1 ^ k2 ^ jnp.uint32(0x1BD11BDA))
    i = pl.program_id(0)
    row0 = lax.convert_element_type(i * br, jnp.uint32)
    r_iota = lax.broadcasted_iota(jnp.uint32, (br, ncols), 0)
    c_iota = lax.broadcasted_iota(jnp.uint32, (br, ncols), 1)
    p = (row0 + r_iota) * jnp.uint32(rmul) + c_iota * jnp.uint32(cmul) \
        + jnp.uint32(poff)
    x0, x1 = _threefry2x32(jnp.zeros((br, ncols), jnp.uint32), p, ks)
    o_ref[...] = _uniform_from_bits(x0 ^ x1, minval, scale)


def _gen_noise(nrows, ncols, br, rmul, cmul, mag, poff=0):
    assert nrows % br == 0
    minval = np.float32(-mag)
    scale = np.float32(np.float32(mag) - minval)
    return pl.pallas_call(
        functools.partial(_noise_body, br=br, ncols=ncols, rmul=rmul,
                          cmul=cmul, minval=float(minval), scale=float(scale),
                          poff=poff),
        grid=(nrows // br,),
        out_specs=pl.BlockSpec((br, ncols), lambda i: (i, 0)),
        out_shape=jax.ShapeDtypeStruct((nrows, ncols), jnp.float32),
    )()


@functools.lru_cache(maxsize=None)
def _sc_noise_fn(nb_sc, nsd, b0, minval, scale):
    info = plsc.get_sparse_core_info()
    NC, NS = info.num_cores, info.num_subcores
    NW = NC * NS
    nchunks = nb_sc // 128
    total_rows = nchunks * nsd
    rows_w = total_rows // NW
    assert nsd % rows_w == 0
    RB = 320
    assert rows_w % RB == 0
    nblk = rows_w // RB
    ks = (0, 42, 42 ^ 0x1BD11BDA)

    mesh = plsc.VectorSubcoreMesh(core_axis_name="c", subcore_axis_name="s")

    @functools.partial(
        pl.kernel,
        out_type=jax.ShapeDtypeStruct((total_rows * 128,), jnp.float32),
        mesh=mesh,
        scratch_types=[pltpu.VMEM((RB * 128,), jnp.float32)],
        compiler_params=pltpu.CompilerParams(use_tc_tiling_on_sc=False),
    )
    def sc_noise(out_hbm, buf):
        wid = lax.axis_index("s") * NC + lax.axis_index("c")
        row0 = wid * rows_w
        jc = lax.convert_element_type(row0 // nsd, jnp.uint32)
        sd0 = lax.convert_element_type(row0 % nsd, jnp.uint32)
        lane16 = lax.iota(jnp.uint32, 16)
        base = (jnp.uint32(b0) + jc * jnp.uint32(128)
                + lane16) * jnp.uint32(nsd) + sd0
        kvec = [jnp.full((16,), jnp.uint32(k)) for k in ks]

        def row_body(t, _):
            for m in range(8):
                p = base + jnp.uint32(m * 16 * nsd) + lax.convert_element_type(
                    t, jnp.uint32)
                x0, x1 = _threefry2x32(jnp.zeros((16,), jnp.uint32), p,
                                       kvec)
                v = _uniform_from_bits(x0 ^ x1, minval, scale)
                tl = t % RB
                buf[pl.ds(tl * 128 + m * 16, 16)] = v
            return 0

        def blk_body(blk, _):
            lax.fori_loop(blk * RB, (blk + 1) * RB, row_body, 0)
            pltpu.sync_copy(
                buf, out_hbm.at[pl.ds((row0 + blk * RB) * 128, RB * 128)])
            return 0

        lax.fori_loop(0, nblk, blk_body, 0)

    return sc_noise


def _add_t_body(e_ref, n_ref, o_ref, *, cb, nsd):
    eb = e_ref[...].reshape(cb, nsd)
    o_ref[...] = jnp.swapaxes(eb, 0, 1) + n_ref[...]


def _add_t(e2, n2, cb, nb):
    nsd, nb_tc = n2.shape
    erows = cb * nsd // 128
    return pl.pallas_call(
        functools.partial(_add_t_body, cb=cb, nsd=nsd),
        grid=(nb_tc // cb,),
        in_specs=[
            pl.BlockSpec((erows, 128), lambda i: (i, 0)),
            pl.BlockSpec((nsd, cb), lambda i: (0, i)),
        ],
        out_specs=pl.BlockSpec((nsd, cb), lambda i: (0, i)),
        out_shape=jax.ShapeDtypeStruct((nsd, nb), jnp.float32),
    )(e2, n2)


def _add_t_fill_body(big_ref, e_ref, n_ref, o_ref, *, cb, nsd):
    eb = e_ref[...].reshape(cb, nsd)
    o_ref[...] = jnp.swapaxes(eb, 0, 1) + n_ref[...]


def _add_t_fill(big, e2, nsc, cb, col0):
    nsd, nb = big.shape
    erows = cb * nsd // 128
    blk0 = col0 // cb
    return pl.pallas_call(
        functools.partial(_add_t_fill_body, cb=cb, nsd=nsd),
        grid=((nb - col0) // cb,),
        in_specs=[
            pl.BlockSpec(memory_space=pl.ANY),
            pl.BlockSpec((erows, 128), lambda i: (i + blk0, 0)),
            pl.BlockSpec((nsd, cb), lambda i: (i, 0)),
        ],
        out_specs=pl.BlockSpec((nsd, cb), lambda i: (0, i + blk0)),
        out_shape=jax.ShapeDtypeStruct((nsd, nb), jnp.float32),
        input_output_aliases={0: 0},
    )(big, e2, nsc)


def kernel(input, table):
    bsz, seq = input.shape
    V, D = table.shape
    ntok = bsz * seq
    mag = np.float32(_NOISE_ALPHA) / np.float32(np.sqrt(np.float32(seq * D)))

    ids2d = input.reshape(ntok // 128, 128).astype(jnp.int32)
    embed = _sc_gather_fn(V, D, ntok)(table, ids2d)

    nsd = seq * D
    minval = np.float32(-mag)
    scale = np.float32(np.float32(mag) - minval)
    NB_SC = 1024
    nb_tc = bsz - NB_SC
    noise = _gen_noise(nsd, nb_tc, br=256, rmul=1, cmul=nsd, mag=mag)
    nsc_flat = _sc_noise_fn(NB_SC, nsd, nb_tc, float(minval), float(scale))()
    nsc = nsc_flat.reshape(NB_SC // 128 * nsd, 128)
    e2 = embed.reshape(ntok * D // 128, 128)
    out_a = _add_t(e2, noise, cb=128, nb=bsz)
    out_t = _add_t_fill(out_a, e2, nsc, cb=128, col0=nb_tc)
    return out_t.reshape(seq, D, bsz).transpose(2, 0, 1)
